# Initial kernel scaffold; baseline (speedup 1.0000x reference)
import jax
import jax.numpy as jnp
from jax.experimental import pallas as pl


def kernel(input, hidden, embedding, W, b):
    raise NotImplementedError("write your pallas kernel here")


# speedup vs baseline: 2.9184x; 2.9184x over previous
import jax
import jax.numpy as jnp
from jax import lax
from jax.experimental import pallas as pl
from jax.experimental.pallas import tpu as pltpu
from jax.experimental.pallas import tpu_sc as plsc

VOCAB = 1000000
HIDDEN = 64
N_IDX = 16384 * 200
NC, NS = 2, 16
NW = NC * NS
PER_W = N_IDX // NW
CHUNK = 128
K = 8
ROUNDS = PER_W // (CHUNK * K)
MV_BLK = 25600
P_LEN = 1024000


def _matvec_body(e_ref, w_ref, p_ref):
    p_ref[...] = jax.lax.dot_general(
        w_ref[...], e_ref[...],
        dimension_numbers=(((1,), (1,)), ((), ())),
        preferred_element_type=jnp.float32)


def _matvec(embedding, W):
    return pl.pallas_call(
        _matvec_body,
        grid=(P_LEN // MV_BLK,),
        in_specs=[
            pl.BlockSpec((MV_BLK, HIDDEN), lambda i: (i, 0)),
            pl.BlockSpec((1, HIDDEN), lambda i: (0, 0)),
        ],
        out_specs=pl.BlockSpec((1, MV_BLK), lambda i: (0, i)),
        out_shape=jax.ShapeDtypeStruct((1, P_LEN), jnp.float32),
    )(embedding, W)


GROUPS = PER_W // (K * CHUNK)


def _gather_sum_body(p_hbm, idx_hbm, out_hbm, idx_v, gbuf, ovec, sem0, sem1):
    wid = lax.axis_index("s") * NC + lax.axis_index("c")
    base = wid * PER_W
    sems = (sem0, sem1)
    gsz = K * CHUNK

    pltpu.sync_copy(idx_hbm.at[pl.ds(base, PER_W)], idx_v)

    def fire(g, s):
        for k in range(K):
            o = pl.multiple_of(g * gsz + k * CHUNK, CHUNK)
            pltpu.async_copy(
                p_hbm.at[idx_v.at[pl.ds(o, CHUNK)]],
                gbuf.at[pl.ds(s * gsz + k * CHUNK, CHUNK)],
                sems[s])

    def drain(s):
        for k in range(K):
            pltpu.make_async_copy(
                p_hbm.at[idx_v.at[pl.ds(0, CHUNK)]],
                gbuf.at[pl.ds(s * gsz + k * CHUNK, CHUNK)],
                sems[s]).wait()

    def accum(s, accs):
        accs = list(accs)
        for j in range(gsz // 16):
            v = gbuf[pl.ds(s * gsz + j * 16, 16)]
            accs[j % 8] = accs[j % 8] + v
        return tuple(accs)

    fire(0, 0)
    fire(1, 1)

    def body(t, accs):
        drain(0)
        accs = accum(0, accs)

        @pl.when(t < GROUPS // 2 - 1)
        def _():
            fire(2 * t + 2, 0)

        drain(1)
        accs = accum(1, accs)

        @pl.when(t < GROUPS // 2 - 1)
        def _():
            fire(2 * t + 3, 1)

        return accs

    zeros = jnp.zeros((16,), jnp.float32)
    accs = lax.fori_loop(0, GROUPS // 2, body, (zeros,) * 8)

    total = accs[0]
    for a in accs[1:]:
        total = total + a
    ovec[...] = total
    pltpu.sync_copy(ovec, out_hbm.at[wid])


def _gather_sum(p, idx):
    mesh = plsc.VectorSubcoreMesh(core_axis_name="c", subcore_axis_name="s",
                                  num_cores=NC, num_subcores=NS)
    f = pl.kernel(
        _gather_sum_body,
        out_type=jax.ShapeDtypeStruct((NW, 16), jnp.float32),
        mesh=mesh,
        scratch_types=[
            pltpu.VMEM((PER_W,), jnp.int32),
            pltpu.VMEM((2 * K * CHUNK,), jnp.float32),
            pltpu.VMEM((16,), jnp.float32),
            pltpu.SemaphoreType.DMA,
            pltpu.SemaphoreType.DMA,
        ],
    )
    return f(p, idx)


def _finish_body(part_ref, b_ref, o1_ref, o2_ref):
    s = jnp.sum(part_ref[...]) + b_ref[0, 0]
    v = jnp.reshape(jax.nn.sigmoid(s), (1, 1))
    o1_ref[...] = v
    o2_ref[...] = v


def _finish(partials, b):
    return pl.pallas_call(
        _finish_body,
        out_shape=(jax.ShapeDtypeStruct((1, 1), jnp.float32),
                   jax.ShapeDtypeStruct((1, 1), jnp.float32)),
    )(partials, b)


def kernel(input, hidden, embedding, W, b):
    idx = input.reshape(-1).astype(jnp.int32)
    p = _matvec(embedding, W).reshape(P_LEN)
    partials = _gather_sum(p, idx)
    output, hidden_out = _finish(partials, b.reshape(1, 1))
    return (output, hidden_out)


# speedup vs baseline: 9.7010x; 3.3241x over previous
import jax
import jax.numpy as jnp
from jax import lax
from jax.experimental import pallas as pl
from jax.experimental.pallas import tpu as pltpu
from jax.experimental.pallas import tpu_sc as plsc

VOCAB = 1000000
HIDDEN = 64
N_IDX = 16384 * 200
N_HALF = N_IDX // 2
NC, NS = 2, 16
NW = NC * NS
CHUNK = 128
CPW = N_HALF // (NW * CHUNK)
BATCH = 16
DEPTH = 2
NBATCH = CPW // BATCH
SLICE = 1024000 // NS
MV_BLK = 25600
P_LEN = 1024000


def _matvec_body(e_ref, w_ref, p_ref):
    i = pl.program_id(0)
    s = jax.lax.dot_general(
        w_ref[...], e_ref[...],
        dimension_numbers=(((1,), (0,)), ((), ())),
        preferred_element_type=jnp.float32)
    col = i * MV_BLK + jax.lax.broadcasted_iota(jnp.int32, (1, MV_BLK), 1)
    p_ref[...] = jnp.where(col < VOCAB, s, 0.0)


def _matvec(embedding_t, W):
    return pl.pallas_call(
        _matvec_body,
        grid=(P_LEN // MV_BLK,),
        in_specs=[
            pl.BlockSpec((HIDDEN, MV_BLK), lambda i: (0, i)),
            pl.BlockSpec((1, HIDDEN), lambda i: (0, 0)),
        ],
        out_specs=pl.BlockSpec((1, MV_BLK), lambda i: (0, i)),
        out_shape=jax.ShapeDtypeStruct((1, P_LEN), jnp.float32),
    )(embedding_t, W)


def _hist_body(idx_hbm, out_hbm, idx_v, ones_v, zbuf, counts, sem):
    cid = lax.axis_index("c")
    sid = lax.axis_index("s")
    wid = sid * NC + cid

    ones16 = jnp.ones((16,), jnp.float32)
    zeros16 = jnp.zeros((16,), jnp.float32)
    for i in range(CHUNK // 16):
        ones_v[pl.ds(i * 16, 16)] = ones16
    for i in range(zbuf.shape[0] // 16):
        zbuf[pl.ds(i * 16, 16)] = zeros16

    zn = zbuf.shape[0]
    for j in range(SLICE // zn):
        pltpu.sync_copy(zbuf, counts.at[pl.ds(sid * SLICE + j * zn, zn)])

    pltpu.sync_copy(idx_hbm.at[pl.ds(wid * CPW * CHUNK, CPW * CHUNK)], idx_v)

    plsc.subcore_barrier()

    def fire(batch):
        for j in range(BATCH):
            o = pl.multiple_of((batch * BATCH + j) * CHUNK, CHUNK)
            pltpu.async_copy(ones_v, counts.at[idx_v.at[pl.ds(o, CHUNK)]],
                             sem, add=True)

    def drain():
        for j in range(BATCH):
            pltpu.make_async_copy(ones_v, counts.at[idx_v.at[pl.ds(0, CHUNK)]],
                                  sem).wait()

    for b in range(DEPTH):
        fire(b)

    def body(t, carry):
        drain()

        @pl.when(t < NBATCH - DEPTH)
        def _():
            fire(t + DEPTH)

        return carry

    lax.fori_loop(0, NBATCH - 1, body, 0)
    drain()

    plsc.subcore_barrier()

    pltpu.sync_copy(counts.at[pl.ds(sid * SLICE, SLICE)],
                    out_hbm.at[pl.ds(cid * P_LEN + sid * SLICE, SLICE)])


def _histogram(idx2):
    mesh = plsc.VectorSubcoreMesh(core_axis_name="c", subcore_axis_name="s",
                                  num_cores=NC, num_subcores=NS)
    f = pl.kernel(
        _hist_body,
        out_type=jax.ShapeDtypeStruct((NC * P_LEN,), jnp.float32),
        mesh=mesh,
        scratch_types=[
            pltpu.VMEM((CPW * CHUNK,), jnp.int32),
            pltpu.VMEM((CHUNK,), jnp.float32),
            pltpu.VMEM((8000,), jnp.float32),
            pltpu.VMEM_SHARED((P_LEN,), jnp.float32),
            pltpu.SemaphoreType.DMA,
        ],
    )
    return f(idx2)


def _finish_body(c1_ref, c2_ref, p_ref, b_ref, o1_ref, o2_ref):
    c = (c1_ref[pl.ds(0, P_LEN)] + c1_ref[pl.ds(P_LEN, P_LEN)]
         + c2_ref[pl.ds(0, P_LEN)] + c2_ref[pl.ds(P_LEN, P_LEN)])
    s = jnp.sum(c * p_ref[0, :]) + b_ref[0, 0]
    v = jnp.reshape(jax.nn.sigmoid(s), (1, 1))
    o1_ref[...] = v
    o2_ref[...] = v


def _finish(counts1, counts2, p, b):
    return pl.pallas_call(
        _finish_body,
        out_shape=(jax.ShapeDtypeStruct((1, 1), jnp.float32),
                   jax.ShapeDtypeStruct((1, 1), jnp.float32)),
    )(counts1, counts2, p, b)


def kernel(input, hidden, embedding, W, b):
    idx2 = input.reshape(2, N_HALF).astype(jnp.int32)
    p = _matvec(embedding.T, W)
    counts1 = _histogram(idx2[0])
    counts2 = _histogram(idx2[1])
    output, hidden_out = _finish(counts1, counts2, p, b.reshape(1, 1))
    return (output, hidden_out)
